# baseline (speedup 1.0000x reference)
import functools

import jax
import jax.numpy as jnp
from jax.experimental import pallas as pl
from jax.experimental.pallas import tpu as pltpu

N_CB = 9
K = 1024
CD = 8
D = 1024
EPS = 1e-12


def _stage(i, resid, in_w_ref, in_b_ref, out_w_ref, out_b_ref, cb_ref, iota_k):
    z_e = jnp.dot(in_w_ref[i], resid) + in_b_ref[i][:, None]
    enc_norm = jnp.sqrt(jnp.sum(z_e * z_e, axis=0, keepdims=True))
    enc_n = z_e / (enc_norm + EPS)
    cb = cb_ref[i]
    cb_norm = jnp.sqrt(jnp.sum(cb * cb, axis=1, keepdims=True))
    cb_n = cb / (cb_norm + EPS)
    twos = jnp.dot(cb_n + cb_n, enc_n)
    s1 = jnp.sum(enc_n * enc_n, axis=0, keepdims=True)
    s2 = jnp.sum(cb_n * cb_n, axis=1, keepdims=True)
    dist = (s1 - twos) + s2
    idx = jnp.argmin(dist, axis=0)
    onehot = (iota_k == idx[None, :]).astype(jnp.float32)
    zq = jnp.dot(cb.T, onehot)
    zq_st = z_e + (zq - z_e)
    zq_out = jnp.dot(out_w_ref[i], zq_st) + out_b_ref[i][:, None]
    return resid - zq_out


def _rvq_kernel(x_ref, in_w_ref, in_b_ref, out_w_ref, out_b_ref, cb_ref,
                out_ref, *, t_tile: int, n_split: int):
    half = t_tile // n_split
    iota_k = jax.lax.broadcasted_iota(jnp.int32, (K, half), 0)
    x = x_ref[0]
    resids = [x[:, h * half:(h + 1) * half] for h in range(n_split)]
    for i in range(N_CB):
        resids = [
            _stage(i, r, in_w_ref, in_b_ref, out_w_ref, out_b_ref, cb_ref,
                   iota_k)
            for r in resids
        ]
    for h in range(n_split):
        out_ref[0, :, h * half:(h + 1) * half] = (
            x[:, h * half:(h + 1) * half] - resids[h])


def kernel(x, in_w, in_b, out_w, out_b, codebooks):
    B, Dd, T = x.shape
    t_tile = 2048
    grid = (B, T // t_tile)

    full = lambda a: pl.BlockSpec(a.shape, lambda b, t: (0,) * a.ndim)
    return pl.pallas_call(
        functools.partial(_rvq_kernel, t_tile=t_tile, n_split=8),
        grid=grid,
        in_specs=[
            pl.BlockSpec((1, Dd, t_tile), lambda b, t: (b, 0, t)),
            full(in_w), full(in_b), full(out_w), full(out_b), full(codebooks),
        ],
        out_specs=pl.BlockSpec((1, Dd, t_tile), lambda b, t: (b, 0, t)),
        out_shape=jax.ShapeDtypeStruct(x.shape, x.dtype),
        compiler_params=pltpu.CompilerParams(
            dimension_semantics=("parallel", "parallel")),
    )(x, in_w, in_b, out_w, out_b, codebooks)


# speedup vs baseline: 1.0474x; 1.0474x over previous
import functools

import jax
import jax.numpy as jnp
from jax.experimental import pallas as pl
from jax.experimental.pallas import tpu as pltpu

N_CB = 9
K = 1024
CD = 8
D = 1024
EPS = 1e-12


def _stage(i, resid, in_w_ref, in_b_ref, out_w_ref, out_b_ref, cb_ref, iota_k):
    z_e = jnp.dot(in_w_ref[i], resid) + in_b_ref[i][:, None]
    enc_norm = jnp.sqrt(jnp.sum(z_e * z_e, axis=0, keepdims=True))
    enc_n = z_e / (enc_norm + EPS)
    cb = cb_ref[i]
    cb_norm = jnp.sqrt(jnp.sum(cb * cb, axis=1, keepdims=True))
    cb_n = cb / (cb_norm + EPS)
    twos = jnp.dot(cb_n + cb_n, enc_n)
    s1 = jnp.sum(enc_n * enc_n, axis=0, keepdims=True)
    s2 = jnp.sum(cb_n * cb_n, axis=1, keepdims=True)
    dist = (s1 - twos) + s2
    idx = jnp.argmin(dist, axis=0)
    onehot = (iota_k == idx[None, :]).astype(jnp.float32)
    zq = jnp.dot(cb.T, onehot)
    zq_st = z_e + (zq - z_e)
    zq_out = jnp.dot(out_w_ref[i], zq_st) + out_b_ref[i][:, None]
    return resid - zq_out


def _rvq_kernel(x_ref, in_w_ref, in_b_ref, out_w_ref, out_b_ref, cb_ref,
                out_ref, *, t_tile: int, n_split: int):
    half = t_tile // n_split
    iota_k = jax.lax.broadcasted_iota(jnp.int32, (K, half), 0)
    x = x_ref[0]
    resids = [x[:, h * half:(h + 1) * half] for h in range(n_split)]
    for i in range(N_CB):
        resids = [
            _stage(i, r, in_w_ref, in_b_ref, out_w_ref, out_b_ref, cb_ref,
                   iota_k)
            for r in resids
        ]
    for h in range(n_split):
        out_ref[0, :, h * half:(h + 1) * half] = (
            x[:, h * half:(h + 1) * half] - resids[h])


def kernel(x, in_w, in_b, out_w, out_b, codebooks):
    B, Dd, T = x.shape
    t_tile = 1024
    grid = (B, T // t_tile)

    full = lambda a: pl.BlockSpec(a.shape, lambda b, t: (0,) * a.ndim)
    return pl.pallas_call(
        functools.partial(_rvq_kernel, t_tile=t_tile, n_split=2),
        grid=grid,
        in_specs=[
            pl.BlockSpec((1, Dd, t_tile), lambda b, t: (b, 0, t)),
            full(in_w), full(in_b), full(out_w), full(out_b), full(codebooks),
        ],
        out_specs=pl.BlockSpec((1, Dd, t_tile), lambda b, t: (b, 0, t)),
        out_shape=jax.ShapeDtypeStruct(x.shape, x.dtype),
        compiler_params=pltpu.CompilerParams(
            dimension_semantics=("parallel", "parallel")),
    )(x, in_w, in_b, out_w, out_b, codebooks)


# speedup vs baseline: 1.2274x; 1.1718x over previous
import functools

import jax
import jax.numpy as jnp
from jax.experimental import pallas as pl
from jax.experimental.pallas import tpu as pltpu

N_CB = 9
K = 1024
CD = 8
D = 1024
EPS = 1e-12


def _stage(i, resid, in_w_ref, in_b_ref, out_w_ref, out_b_ref, cb_ref, iota_k):
    z_e = jnp.dot(in_w_ref[i], resid) + in_b_ref[i][:, None]
    enc_norm = jnp.sqrt(jnp.sum(z_e * z_e, axis=0, keepdims=True))
    enc_n = z_e / (enc_norm + EPS)
    cb = cb_ref[i]
    cb_norm = jnp.sqrt(jnp.sum(cb * cb, axis=1, keepdims=True))
    cb_n = cb / (cb_norm + EPS)
    twos = jnp.dot(cb_n + cb_n, enc_n)
    s1 = jnp.sum(enc_n * enc_n, axis=0, keepdims=True)
    s2 = jnp.sum(cb_n * cb_n, axis=1, keepdims=True)
    dist = (s1 - twos) + s2
    idx = jnp.argmin(dist, axis=0)
    onehot = (iota_k == idx[None, :]).astype(jnp.float32)
    zq = jnp.dot(cb.T, onehot)
    zq_st = z_e + (zq - z_e)
    zq_out = jnp.dot(out_w_ref[i], zq_st) + out_b_ref[i][:, None]
    return resid - zq_out


def _rvq_kernel(x_ref, in_w_ref, in_b_ref, out_w_ref, out_b_ref, cb_ref,
                out_ref, *, t_tile: int, n_split: int):
    half = t_tile // n_split
    iota_k = jax.lax.broadcasted_iota(jnp.int32, (K, half), 0)
    x = x_ref[0]
    resids = [x[:, h * half:(h + 1) * half] for h in range(n_split)]
    for i in range(N_CB):
        resids = [
            _stage(i, r, in_w_ref, in_b_ref, out_w_ref, out_b_ref, cb_ref,
                   iota_k)
            for r in resids
        ]
    for h in range(n_split):
        out_ref[0, :, h * half:(h + 1) * half] = (
            x[:, h * half:(h + 1) * half] - resids[h])


def kernel(x, in_w, in_b, out_w, out_b, codebooks):
    B, Dd, T = x.shape
    t_tile = 1024
    grid = (B, T // t_tile)

    full = lambda a: pl.BlockSpec(a.shape, lambda b, t: (0,) * a.ndim)
    return pl.pallas_call(
        functools.partial(_rvq_kernel, t_tile=t_tile, n_split=4),
        grid=grid,
        in_specs=[
            pl.BlockSpec((1, Dd, t_tile), lambda b, t: (b, 0, t)),
            full(in_w), full(in_b), full(out_w), full(out_b), full(codebooks),
        ],
        out_specs=pl.BlockSpec((1, Dd, t_tile), lambda b, t: (b, 0, t)),
        out_shape=jax.ShapeDtypeStruct(x.shape, x.dtype),
        compiler_params=pltpu.CompilerParams(
            dimension_semantics=("parallel", "parallel")),
    )(x, in_w, in_b, out_w, out_b, codebooks)
